# baseline (speedup 1.0000x reference)
import functools

import jax
import jax.numpy as jnp
from jax import lax
from jax.experimental import pallas as pl
from jax.experimental.pallas import tpu as pltpu
from jax.experimental.pallas import tpu_sc as plsc

N_NODES = 10000
N_EDGES = 320000
A = 128
L = 16
NP = A // 2
CP = 4
NCHUNK = NP // CP
NC = 2
NS = 16
NW = NC * NS
E_PER = N_EDGES // NW
NG = E_PER // L


def _body(ct_hbm, sidx_hbm, didx_hbm, w_hbm, out_hbm,
          si_v, di_v, pmax, wv, sl0, sl1, sem0, sem1):
  cid = lax.axis_index("c")
  sid = lax.axis_index("s")
  wid = sid * NC + cid
  base = wid * E_PER

  pltpu.sync_copy(sidx_hbm.at[pl.ds(base, E_PER)], si_v)
  pltpu.sync_copy(didx_hbm.at[pl.ds(base, E_PER)], di_v)
  pltpu.sync_copy(w_hbm, wv)

  slab = (sl0, sl1)
  sem = (sem0, sem1)

  def start_chunk(c, b):
    pltpu.async_copy(ct_hbm.at[pl.ds(c * CP, CP)], slab[b], sem[b])

  def wait_chunk(c, b):
    pltpu.make_async_copy(ct_hbm.at[pl.ds(c * CP, CP)], slab[b], sem[b]).wait()

  start_chunk(0, 0)
  start_chunk(1, 1)

  for c in range(NCHUNK):
    b = c % 2
    wait_chunk(c, b)
    if c + 2 < NCHUNK:
      start_chunk(c + 2, b)
    sl = slab[b]
    wwin = wv[pl.ds((c * CP // L) * L, L)]
    woff = c * CP - (c * CP // L) * L
    wp = [
        plsc.bitcast(jnp.full((L,), wwin[woff + jj], jnp.int32),
                     jnp.bfloat16)
        for jj in range(CP)
    ]
    first = c == 0

    @pl.loop(0, NG)
    def _grp(grp, sl=sl, wp=wp, first=first):
      sv = si_v[pl.ds(grp * L, L)]
      dv = di_v[pl.ds(grp * L, L)]
      accp = None
      for jj in range(CP):
        row = sl.at[jj]
        s = plsc.bitcast(plsc.load_gather(row, [sv]), jnp.bfloat16)
        d = plsc.bitcast(plsc.load_gather(row, [dv]), jnp.bfloat16)
        m = s * d * wp[jj]
        accp = m if jj == 0 else jnp.maximum(accp, m)
      lo, hi = plsc.unpack(
          accp, format=plsc.PackFormat.INTERLEAVED,
          preferred_element_type=jnp.float32)
      cm = jnp.maximum(lo, hi)
      if not first:
        cm = jnp.maximum(cm, pmax[pl.ds(grp * L, L)])
      pmax[pl.ds(grp * L, L)] = cm

  @pl.loop(0, NG)
  def _sig(i):
    x = pmax[pl.ds(i * L, L)]
    pmax[pl.ds(i * L, L)] = 1.0 / (1.0 + jnp.exp(-x))

  pltpu.sync_copy(pmax, out_hbm.at[pl.ds(base, E_PER)])


@jax.jit
def _run(ct, sidx, didx, w):
  mesh = plsc.VectorSubcoreMesh(
      core_axis_name="c", subcore_axis_name="s", num_cores=NC,
      num_subcores=NS)
  f = pl.kernel(
      _body,
      out_type=jax.ShapeDtypeStruct((N_EDGES,), jnp.float32),
      mesh=mesh,
      compiler_params=pltpu.CompilerParams(
          needs_layout_passes=False, use_tc_tiling_on_sc=False),
      scratch_types=[
          pltpu.VMEM((E_PER,), jnp.int32),
          pltpu.VMEM((E_PER,), jnp.int32),
          pltpu.VMEM((E_PER,), jnp.float32),
          pltpu.VMEM((NP,), jnp.int32),
          pltpu.VMEM((CP, N_NODES), jnp.int32),
          pltpu.VMEM((CP, N_NODES), jnp.int32),
          pltpu.SemaphoreType.DMA,
          pltpu.SemaphoreType.DMA,
      ],
  )
  return f(ct, sidx, didx, w)


def kernel(sparse_codes, edge_index, pattern_weights):
  eidx = edge_index.astype(jnp.int32)
  codes_bf = sparse_codes.astype(jnp.bfloat16)
  ct = jax.lax.bitcast_convert_type(
      codes_bf.T.reshape(NP, 2, N_NODES).transpose(0, 2, 1), jnp.int32)
  w_bf = pattern_weights.astype(jnp.bfloat16)
  w_i32 = jax.lax.bitcast_convert_type(w_bf.reshape(NP, 2), jnp.int32)
  return _run(ct, eidx[0], eidx[1], w_i32)
